# baseline (speedup 1.0000x reference)
import functools
import math

import jax
import jax.numpy as jnp
from jax import lax
from jax.experimental import pallas as pl
from jax.experimental.pallas import tpu as pltpu
from jax.experimental.pallas import tpu_sc as plsc

_DIM = 64
_SCALE = math.sqrt(_DIM)
_LANES = 16
_CHUNK = 512


@functools.lru_cache(maxsize=None)
def _build(n_total: int, vocab: int, dim: int):
    info = plsc.get_sparse_core_info()
    nw = info.num_cores * info.num_subcores
    assert n_total % nw == 0
    n_per_w = n_total // nw
    chunk = min(_CHUNK, n_per_w)
    assert n_per_w % chunk == 0
    n_chunks = n_per_w // chunk

    mesh = plsc.VectorSubcoreMesh(core_axis_name="c", subcore_axis_name="s")

    @functools.partial(
        pl.kernel,
        mesh=mesh,
        compiler_params=pltpu.CompilerParams(use_tc_tiling_on_sc=False),
        out_type=jax.ShapeDtypeStruct((n_total, dim), jnp.float32),
        scratch_types=[
            pltpu.VMEM((chunk,), jnp.int32),
            pltpu.VMEM((chunk, dim), jnp.float32),
            pltpu.SemaphoreType.DMA,
        ],
    )
    def sc_embed(x_hbm, tab_hbm, out_hbm, idx_v, rows_v, sem):
        wid = lax.axis_index("s") * info.num_cores + lax.axis_index("c")
        base = wid * n_per_w

        def chunk_body(ci, carry):
            off = base + ci * chunk
            pltpu.sync_copy(x_hbm.at[pl.ds(off, chunk)], idx_v)
            pltpu.async_copy(tab_hbm.at[idx_v], rows_v, sem).wait()

            def scale_row(r, c2):
                for c in range(dim // _LANES):
                    sl = pl.ds(c * _LANES, _LANES)
                    rows_v[r, sl] = rows_v[r, sl] * _SCALE
                return c2

            lax.fori_loop(0, chunk, scale_row, 0)
            pltpu.sync_copy(rows_v, out_hbm.at[pl.ds(off, chunk)])
            return carry

        lax.fori_loop(0, n_chunks, chunk_body, 0)

    return sc_embed


def kernel(x, table):
    b, s = x.shape
    vocab, dim = table.shape
    n_total = b * s
    fn = _build(n_total, vocab, dim)
    out = fn(x.reshape(n_total).astype(jnp.int32), table)
    return out.reshape(b, s, dim)


# speedup vs baseline: 1.1047x; 1.1047x over previous
import functools
import math

import jax
import jax.numpy as jnp
from jax import lax
from jax.experimental import pallas as pl
from jax.experimental.pallas import tpu as pltpu
from jax.experimental.pallas import tpu_sc as plsc

_DIM = 64
_SCALE = math.sqrt(_DIM)
_LANES = 16
_CHUNK = 512


@functools.lru_cache(maxsize=None)
def _build(n_total: int, vocab: int, dim: int):
    info = plsc.get_sparse_core_info()
    nw = info.num_cores * info.num_subcores
    assert n_total % nw == 0
    n_per_w = n_total // nw
    chunk = min(_CHUNK, n_per_w)
    assert n_per_w % chunk == 0
    n_chunks = n_per_w // chunk

    mesh = plsc.VectorSubcoreMesh(core_axis_name="c", subcore_axis_name="s")

    @functools.partial(
        pl.kernel,
        mesh=mesh,
        compiler_params=pltpu.CompilerParams(use_tc_tiling_on_sc=False),
        out_type=jax.ShapeDtypeStruct((n_total, dim), jnp.float32),
        scratch_types=[
            pltpu.VMEM((n_per_w,), jnp.int32),
            pltpu.VMEM((chunk, dim), jnp.float32),
            pltpu.VMEM((chunk, dim), jnp.float32),
            pltpu.SemaphoreType.DMA,
            pltpu.SemaphoreType.DMA,
            pltpu.SemaphoreType.DMA,
            pltpu.SemaphoreType.DMA,
        ],
    )
    def sc_embed(x_hbm, tab_hbm, out_hbm, idx_v, row0, row1, g0, g1, s0, s1):
        wid = lax.axis_index("s") * info.num_cores + lax.axis_index("c")
        base = wid * n_per_w
        bufs = (row0, row1)
        gsems = (g0, g1)
        ssems = (s0, s1)

        pltpu.sync_copy(x_hbm.at[pl.ds(base, n_per_w)], idx_v)

        def start_gather(ci):
            b = bufs[ci % 2]
            idx = idx_v.at[pl.ds(ci * chunk, chunk)]
            return pltpu.async_copy(tab_hbm.at[idx], b, gsems[ci % 2])

        def scale(b):
            def scale_row(r, c2):
                for c in range(dim // _LANES):
                    sl = pl.ds(c * _LANES, _LANES)
                    b[r, sl] = b[r, sl] * _SCALE
                return c2

            lax.fori_loop(0, chunk, scale_row, 0)

        gathers = [None] * n_chunks
        stores = [None] * n_chunks
        gathers[0] = start_gather(0)
        for ci in range(n_chunks):
            b = bufs[ci % 2]
            if ci + 1 < n_chunks:
                if ci >= 1:
                    stores[ci - 1].wait()
                gathers[ci + 1] = start_gather(ci + 1)
            gathers[ci].wait()
            scale(b)
            stores[ci] = pltpu.async_copy(
                b, out_hbm.at[pl.ds(base + ci * chunk, chunk)], ssems[ci % 2]
            )
        stores[n_chunks - 2].wait()
        stores[n_chunks - 1].wait()

    return sc_embed


def kernel(x, table):
    b, s = x.shape
    vocab, dim = table.shape
    n_total = b * s
    fn = _build(n_total, vocab, dim)
    out = fn(x.reshape(n_total).astype(jnp.int32), table)
    return out.reshape(b, s, dim)
